# Initial kernel scaffold; baseline (speedup 1.0000x reference)
import jax
import jax.numpy as jnp
from jax.experimental import pallas as pl


def kernel(input_char):
    raise NotImplementedError("write your pallas kernel here")


# speedup vs baseline: 1.5961x; 1.5961x over previous
import functools

import jax
import jax.numpy as jnp
from jax import lax
from jax.experimental import pallas as pl
from jax.experimental.pallas import tpu as pltpu
from jax.experimental.pallas import tpu_sc as plsc

ONEHOT = 38
BATCH = 16384

_INFO = plsc.get_sparse_core_info()
NC = _INFO.num_cores
NS = _INFO.num_subcores
LANES = _INFO.num_lanes
NW = NC * NS
BPW = BATCH // NW
WORDS = BPW * ONEHOT

_mesh = plsc.VectorSubcoreMesh(core_axis_name="c", subcore_axis_name="s")


@functools.partial(
    pl.kernel,
    mesh=_mesh,
    out_type=jax.ShapeDtypeStruct((BATCH * ONEHOT,), jnp.float32),
    scratch_types=[
        pltpu.VMEM((BPW,), jnp.int32),
        pltpu.VMEM((WORDS,), jnp.float32),
    ],
    compiler_params=pltpu.CompilerParams(needs_layout_passes=False),
)
def _onehot_sc(idx_hbm, out_hbm, idx_v, rows_v):
    wid = lax.axis_index("s") * NC + lax.axis_index("c")
    base = wid * BPW

    pltpu.sync_copy(idx_hbm.at[pl.ds(base, BPW)], idx_v)

    zeros = jnp.zeros((LANES,), jnp.float32)

    def zero_body(i, carry):
        rows_v[pl.ds(i * LANES, LANES)] = zeros
        return carry

    lax.fori_loop(0, WORDS // LANES, zero_body, 0, unroll=8)

    ones = jnp.ones((LANES,), jnp.float32)
    lane = lax.iota(jnp.int32, LANES)

    def scatter_body(g, carry):
        idxs = idx_v[pl.ds(g * LANES, LANES)]
        flat = (g * LANES + lane) * ONEHOT + idxs
        plsc.store_scatter(rows_v, [flat], ones)
        return carry

    lax.fori_loop(0, BPW // LANES, scatter_body, 0)

    pltpu.sync_copy(rows_v, out_hbm.at[pl.ds(base * ONEHOT, WORDS)])


def kernel(input_char):
    flat = _onehot_sc(input_char.astype(jnp.int32))
    return flat.reshape(BATCH, ONEHOT)


# speedup vs baseline: 2.2011x; 1.3791x over previous
import functools

import jax
import jax.numpy as jnp
from jax import lax
from jax.experimental import pallas as pl
from jax.experimental.pallas import tpu as pltpu
from jax.experimental.pallas import tpu_sc as plsc

ONEHOT = 38
BATCH = 16384

_INFO = plsc.get_sparse_core_info()
NC = _INFO.num_cores
NS = _INFO.num_subcores
LANES = _INFO.num_lanes
NW = NC * NS
BPW = BATCH // NW
WORDS = BPW * ONEHOT

_mesh = plsc.VectorSubcoreMesh(core_axis_name="c", subcore_axis_name="s")


@functools.partial(
    pl.kernel,
    mesh=_mesh,
    out_type=jax.ShapeDtypeStruct((BATCH, ONEHOT), jnp.float32),
    scratch_types=[
        pltpu.VMEM((BPW,), jnp.int32),
        pltpu.VMEM((BPW, ONEHOT), jnp.float32),
    ],
    compiler_params=pltpu.CompilerParams(needs_layout_passes=False),
)
def _onehot_sc(idx_hbm, out_hbm, idx_v, rows_v):
    wid = lax.axis_index("s") * NC + lax.axis_index("c")
    base = wid * BPW

    pltpu.sync_copy(idx_hbm.at[pl.ds(base, BPW)], idx_v)

    zeros = jnp.zeros((LANES,), jnp.float32)

    def zero_body(r, carry):
        rows_v[r, pl.ds(0, LANES)] = zeros
        rows_v[r, pl.ds(LANES, LANES)] = zeros
        rows_v[r, pl.ds(ONEHOT - LANES, LANES)] = zeros
        return carry

    lax.fori_loop(0, BPW, zero_body, 0, unroll=4)

    ones = jnp.ones((LANES,), jnp.float32)
    lane = lax.iota(jnp.int32, LANES)

    def scatter_body(g, carry):
        cols = idx_v[pl.ds(g * LANES, LANES)]
        rows = g * LANES + lane
        plsc.store_scatter(rows_v, [rows, cols], ones)
        return carry

    lax.fori_loop(0, BPW // LANES, scatter_body, 0)

    pltpu.sync_copy(rows_v, out_hbm.at[pl.ds(base, BPW)])


def kernel(input_char):
    return _onehot_sc(input_char.astype(jnp.int32))


# speedup vs baseline: 3.0393x; 1.3808x over previous
import functools

import jax
import jax.numpy as jnp
from jax import lax
from jax.experimental import pallas as pl
from jax.experimental.pallas import tpu as pltpu
from jax.experimental.pallas import tpu_sc as plsc

ONEHOT = 38
BATCH = 16384

_INFO = plsc.get_sparse_core_info()
NC = _INFO.num_cores
NS = _INFO.num_subcores
LANES = _INFO.num_lanes
NW = NC * NS
CPW = BATCH // NW

_mesh = plsc.VectorSubcoreMesh(core_axis_name="c", subcore_axis_name="s")


@functools.partial(
    pl.kernel,
    mesh=_mesh,
    out_type=jax.ShapeDtypeStruct((ONEHOT, BATCH), jnp.float32),
    scratch_types=[
        pltpu.VMEM((CPW,), jnp.int32),
        pltpu.VMEM((ONEHOT, CPW), jnp.float32),
    ],
    compiler_params=pltpu.CompilerParams(needs_layout_passes=False),
)
def _onehot_sc(idx_hbm, out_hbm, idx_v, blk_v):
    wid = lax.axis_index("s") * NC + lax.axis_index("c")
    base = wid * CPW

    pltpu.sync_copy(idx_hbm.at[pl.ds(base, CPW)], idx_v)

    zeros = jnp.zeros((LANES,), jnp.float32)

    def zero_body(r, carry):
        def zero_inner(c, carry2):
            blk_v[r, pl.ds(c * LANES, LANES)] = zeros
            return carry2

        return lax.fori_loop(0, CPW // LANES, zero_inner, carry, unroll=8)

    lax.fori_loop(0, ONEHOT, zero_body, 0)

    ones = jnp.ones((LANES,), jnp.float32)
    lane = lax.iota(jnp.int32, LANES)

    def scatter_body(g, carry):
        rows = idx_v[pl.ds(g * LANES, LANES)]
        cols = g * LANES + lane
        plsc.store_scatter(blk_v, [rows, cols], ones)
        return carry

    lax.fori_loop(0, CPW // LANES, scatter_body, 0)

    pltpu.sync_copy(blk_v, out_hbm.at[:, pl.ds(base, CPW)])


def kernel(input_char):
    return _onehot_sc(input_char.astype(jnp.int32)).T
